# Initial kernel scaffold; baseline (speedup 1.0000x reference)
import jax
import jax.numpy as jnp
from jax.experimental import pallas as pl


def kernel(x, edge_index, W0, b0, W1, b1, W2, b2, ln0_g, ln0_b, ln1_g, ln1_b, ln2_g, ln2_b, P0W, P0b, P1W, P1b, P2W, P2b, inW, inb, fc1W, fc1b, fc2W, fc2b):
    raise NotImplementedError("write your pallas kernel here")


# speedup vs baseline: 10.5463x; 10.5463x over previous
import functools

import jax
import jax.numpy as jnp
from jax import lax
from jax.experimental import pallas as pl
from jax.experimental.pallas import tpu as pltpu
from jax.experimental.pallas import tpu_sc as plsc

N = 10000
D = 128
C = 40
E = 320000
NPAD = 10240

NC, NS = 2, 16
NW = NC * NS


DEG_TPW = NPAD // NS
DEG_EPW = E // NS
DEG_CH = 80
DEG_NCH = DEG_EPW // DEG_CH

_sc_mesh = plsc.VectorSubcoreMesh(
    core_axis_name="c", subcore_axis_name="s", num_cores=NC, num_subcores=NS)


def _degree_body(dst_hbm, out_hbm, idxv, onesv, acc):
    c = lax.axis_index("c")
    s = lax.axis_index("s")

    @pl.when(c == 0)
    def _():
        def fill(j, carry):
            onesv[pl.ds(j * 16, 16)] = jnp.full((16,), 1.0, jnp.float32)
            return carry
        lax.fori_loop(0, DEG_TPW // 16, fill, 0)
        pltpu.sync_copy(onesv, acc.at[pl.ds(s * DEG_TPW, DEG_TPW)])
        plsc.subcore_barrier()
        base = s * DEG_EPW

        def body(k, carry):
            pltpu.sync_copy(dst_hbm.at[pl.ds(base + k * DEG_CH, DEG_CH)], idxv)
            pltpu.sync_copy(onesv.at[pl.ds(0, DEG_CH)], acc.at[idxv], add=True)
            return carry
        lax.fori_loop(0, DEG_NCH, body, 0)
        plsc.subcore_barrier()
        pltpu.sync_copy(acc.at[pl.ds(s * DEG_TPW, DEG_TPW)],
                        out_hbm.at[pl.ds(s * DEG_TPW, DEG_TPW)])


_degree = pl.kernel(
    _degree_body,
    out_type=jax.ShapeDtypeStruct((NPAD,), jnp.float32),
    mesh=_sc_mesh,
    scratch_types=[
        pltpu.VMEM((DEG_CH,), jnp.int32),
        pltpu.VMEM((DEG_TPW,), jnp.float32),
        pltpu.VMEM_SHARED((NPAD,), jnp.float32),
    ],
)


AGG_EPW = E // NW
AGG_CH = 80
AGG_NCH = AGG_EPW // AGG_CH
AGG_RPT = 624
AGG_TAIL = N - AGG_RPT * NS


def _seed_rows(src_hbm, acc, s):
    pltpu.sync_copy(src_hbm.at[pl.ds(s * AGG_RPT, AGG_RPT), :],
                    acc.at[pl.ds(s * AGG_RPT, AGG_RPT), :])

    @pl.when(s == NS - 1)
    def _():
        pltpu.sync_copy(src_hbm.at[pl.ds(AGG_RPT * NS, AGG_TAIL), :],
                        acc.at[pl.ds(AGG_RPT * NS, AGG_TAIL), :])


def _agg_body(y_hbm, z_hbm, src_hbm, dst_hbm, out_hbm,
              srcv, dstv, rows, acc, gsem):
    c = lax.axis_index("c")
    s = lax.axis_index("s")
    wid = s * NC + c

    @pl.when(c == 0)
    def _():
        _seed_rows(y_hbm, acc, s)

    @pl.when(c == 1)
    def _():
        _seed_rows(z_hbm, acc, s)

    plsc.subcore_barrier()
    base = wid * AGG_EPW

    def body(k, carry):
        off = base + k * AGG_CH
        pltpu.sync_copy(src_hbm.at[pl.ds(off, AGG_CH)], srcv)
        pltpu.sync_copy(dst_hbm.at[pl.ds(off, AGG_CH)], dstv)
        pltpu.async_copy(y_hbm.at[srcv], rows, gsem).wait()
        pltpu.sync_copy(rows, acc.at[dstv], add=True)
        return carry
    lax.fori_loop(0, AGG_NCH, body, 0)

    plsc.subcore_barrier()
    pltpu.sync_copy(acc.at[pl.ds(s * AGG_RPT, AGG_RPT), :],
                    out_hbm.at[c, pl.ds(s * AGG_RPT, AGG_RPT), :])

    @pl.when(s == NS - 1)
    def _():
        pltpu.sync_copy(acc.at[pl.ds(AGG_RPT * NS, AGG_TAIL), :],
                        out_hbm.at[c, pl.ds(AGG_RPT * NS, AGG_TAIL), :])


_aggregate = pl.kernel(
    _agg_body,
    out_type=jax.ShapeDtypeStruct((NC, N, D), jnp.float32),
    mesh=_sc_mesh,
    scratch_types=[
        pltpu.VMEM((AGG_CH,), jnp.int32),
        pltpu.VMEM((AGG_CH,), jnp.int32),
        pltpu.VMEM((AGG_CH, D), jnp.float32),
        pltpu.VMEM_SHARED((N, D), jnp.float32),
        pltpu.SemaphoreType.DMA,
    ],
)


RB = 1000
GRID = N // RB

_row = pl.BlockSpec((RB, D), lambda i: (i, 0))
_col = pl.BlockSpec((RB, 1), lambda i: (i, 0))
_wmat = pl.BlockSpec((D, D), lambda i: (0, 0))
_brow = pl.BlockSpec((1, D), lambda i: (0, 0))
_agg2 = pl.BlockSpec((NC, RB, D), lambda i: (0, i, 0))
_f32 = jnp.float32


def _gelu(t):
    return 0.5 * t * (1.0 + lax.erf(t * 0.7071067811865476))


def _ln_gelu(t, g, b):
    mu = jnp.mean(t, axis=1, keepdims=True)
    var = jnp.mean((t - mu) ** 2, axis=1, keepdims=True)
    ln = (t - mu) * lax.rsqrt(var + 1e-5) * g + b
    return _gelu(ln)


def _tc_pre_body(x_ref, deg_ref, w_ref, pw_ref, inw_ref, pb_ref, inb_ref,
                 y_ref, q_ref, s_ref):
    xb = x_ref[...]
    dis = lax.rsqrt(deg_ref[...])
    y_ref[...] = jnp.dot(xb, w_ref[...], preferred_element_type=_f32) * dis
    q_ref[...] = jnp.dot(xb, pw_ref[...], preferred_element_type=_f32) + pb_ref[...]
    s_ref[...] = jnp.dot(xb, inw_ref[...], preferred_element_type=_f32) + inb_ref[...]


_tc_pre = pl.pallas_call(
    _tc_pre_body,
    grid=(GRID,),
    in_specs=[_row, _col, _wmat, _wmat, _wmat, _brow, _brow],
    out_specs=[_row, _row, _row],
    out_shape=[jax.ShapeDtypeStruct((N, D), _f32)] * 3,
)


def _tc_mid_body(p_ref, q_ref, deg_ref, b_ref, g_ref, lb_ref,
                 w_ref, pw_ref, pb_ref, y_ref, qn_ref):
    dis = lax.rsqrt(deg_ref[...])
    u = (p_ref[0] + p_ref[1]) * dis + b_ref[...] + q_ref[...]
    gl = _ln_gelu(u, g_ref[...], lb_ref[...])
    y_ref[...] = jnp.dot(gl, w_ref[...], preferred_element_type=_f32) * dis
    qn_ref[...] = jnp.dot(u, pw_ref[...], preferred_element_type=_f32) + pb_ref[...]


_tc_mid = pl.pallas_call(
    _tc_mid_body,
    grid=(GRID,),
    in_specs=[_agg2, _row, _col, _brow, _brow, _brow, _wmat, _wmat, _brow],
    out_specs=[_row, _row],
    out_shape=[jax.ShapeDtypeStruct((N, D), _f32)] * 2,
)


def _tc_head_body(p_ref, q_ref, deg_ref, s_ref, b_ref, g_ref, lb_ref,
                  fc1w_ref, fc1b_ref, fc2w_ref, fc2b_ref, o_ref):
    dis = lax.rsqrt(deg_ref[...])
    u = (p_ref[0] + p_ref[1]) * dis + b_ref[...] + q_ref[...]
    gl = _ln_gelu(u, g_ref[...], lb_ref[...])
    z = s_ref[...] + gl
    h = _gelu(jnp.dot(z, fc1w_ref[...], preferred_element_type=_f32)
              + fc1b_ref[...])
    logits = jnp.dot(h, fc2w_ref[...], preferred_element_type=_f32) + fc2b_ref[...]
    m = jnp.max(logits, axis=1, keepdims=True)
    lse = jnp.log(jnp.sum(jnp.exp(logits - m), axis=1, keepdims=True))
    o_ref[...] = logits - m - lse


_tc_head = pl.pallas_call(
    _tc_head_body,
    grid=(GRID,),
    in_specs=[_agg2, _row, _col, _row, _brow, _brow, _brow,
              _wmat, _brow,
              pl.BlockSpec((D, C), lambda i: (0, 0)),
              pl.BlockSpec((1, C), lambda i: (0, 0))],
    out_specs=pl.BlockSpec((RB, C), lambda i: (i, 0)),
    out_shape=jax.ShapeDtypeStruct((N, C), _f32),
)


def kernel(x, edge_index, W0, b0, W1, b1, W2, b2, ln0_g, ln0_b, ln1_g, ln1_b,
           ln2_g, ln2_b, P0W, P0b, P1W, P1b, P2W, P2b, inW, inb,
           fc1W, fc1b, fc2W, fc2b):
    src = edge_index[0]
    dst = edge_index[1]
    r = lambda v: v.reshape(1, -1)

    deg = _degree(dst)
    deg_col = deg[:N].reshape(N, 1)
    zeros = jnp.zeros((N, D), _f32)

    y0, q0, s = _tc_pre(x, deg_col, W0, P0W, inW, r(P0b), r(inb))
    p0 = _aggregate(y0, zeros, src, dst)
    y1, q1 = _tc_mid(p0, q0, deg_col, r(b0), r(ln0_g), r(ln0_b),
                     W1, P1W, r(P1b))
    p1 = _aggregate(y1, zeros, src, dst)
    y2, q2 = _tc_mid(p1, q1, deg_col, r(b1), r(ln1_g), r(ln1_b),
                     W2, P2W, r(P2b))
    p2 = _aggregate(y2, zeros, src, dst)
    return _tc_head(p2, q2, deg_col, s, r(b2), r(ln2_g), r(ln2_b),
                    fc1W, r(fc1b), fc2W, r(fc2b))
